# baseline (speedup 1.0000x reference)
import jax
import jax.numpy as jnp
from jax.experimental import pallas as pl
from jax.experimental.pallas import tpu as pltpu


def _support_kernel(x_ref, w_ref, out_ref):
    out_ref[...] = jnp.dot(x_ref[...], w_ref[...],
                           preferred_element_type=jnp.float32)


def _gcn_kernel(adj_ref, support_ref, x_ref, wself_ref, b_ref, out_ref):
    agg = jnp.dot(adj_ref[...].astype(jnp.bfloat16),
                  support_ref[...].astype(jnp.bfloat16),
                  preferred_element_type=jnp.float32)
    self_part = jnp.dot(x_ref[...], wself_ref[...],
                        preferred_element_type=jnp.float32)
    out_ref[...] = agg + self_part + b_ref[...]


def kernel(input, adj, adj_homo, W, W_self, b):
    x = input.astype(jnp.float32)
    adj_homo = adj_homo.astype(jnp.float32)
    N, din = x.shape
    dout = W.shape[1]
    b2d = b.reshape(1, dout).astype(jnp.float32)

    support = pl.pallas_call(
        _support_kernel,
        out_shape=jax.ShapeDtypeStruct((N, dout), jnp.float32),
    )(x, W.astype(jnp.float32))

    BM = 200
    nm = N // BM

    out = pl.pallas_call(
        _gcn_kernel,
        grid=(nm,),
        in_specs=[
            pl.BlockSpec((BM, N), lambda m: (m, 0)),
            pl.BlockSpec((N, dout), lambda m: (0, 0)),
            pl.BlockSpec((BM, din), lambda m: (m, 0)),
            pl.BlockSpec((din, dout), lambda m: (0, 0)),
            pl.BlockSpec((1, dout), lambda m: (0, 0)),
        ],
        out_specs=pl.BlockSpec((BM, dout), lambda m: (m, 0)),
        out_shape=jax.ShapeDtypeStruct((N, dout), jnp.float32),
        compiler_params=pltpu.CompilerParams(
            dimension_semantics=("parallel",)),
    )(adj_homo, support, x, W_self.astype(jnp.float32), b2d)
    return out


# speedup vs baseline: 1.0693x; 1.0693x over previous
import jax
import jax.numpy as jnp
from jax.experimental import pallas as pl
from jax.experimental.pallas import tpu as pltpu


def _gcn_kernel(adj_ref, x_ref, w_ref, wself_ref, b_ref, out_ref,
                support_ref, *, bm):
    m = pl.program_id(0)

    @pl.when(m == 0)
    def _():
        support_ref[...] = jnp.dot(x_ref[...], w_ref[...],
                                   preferred_element_type=jnp.float32)

    x_blk = x_ref[pl.ds(m * bm, bm), :]
    agg = jnp.dot(adj_ref[...], support_ref[...],
                  preferred_element_type=jnp.float32)
    self_part = jnp.dot(x_blk, wself_ref[...],
                        preferred_element_type=jnp.float32)
    out_ref[...] = agg + self_part + b_ref[...]


def kernel(input, adj, adj_homo, W, W_self, b):
    x = input.astype(jnp.float32)
    adj_homo = adj_homo.astype(jnp.float32)
    N, din = x.shape
    dout = W.shape[1]
    b2d = b.reshape(1, dout).astype(jnp.float32)

    BM = 200
    nm = N // BM

    import functools
    out = pl.pallas_call(
        functools.partial(_gcn_kernel, bm=BM),
        grid=(nm,),
        in_specs=[
            pl.BlockSpec((BM, N), lambda m: (m, 0)),
            pl.BlockSpec((N, din), lambda m: (0, 0)),
            pl.BlockSpec((din, dout), lambda m: (0, 0)),
            pl.BlockSpec((din, dout), lambda m: (0, 0)),
            pl.BlockSpec((1, dout), lambda m: (0, 0)),
        ],
        out_specs=pl.BlockSpec((BM, dout), lambda m: (m, 0)),
        out_shape=jax.ShapeDtypeStruct((N, dout), jnp.float32),
        scratch_shapes=[pltpu.VMEM((N, dout), jnp.float32)],
        compiler_params=pltpu.CompilerParams(
            dimension_semantics=("arbitrary",)),
    )(adj_homo, x, W.astype(jnp.float32), W_self.astype(jnp.float32), b2d)
    return out


# speedup vs baseline: 1.0749x; 1.0053x over previous
import jax
import jax.numpy as jnp
from jax.experimental import pallas as pl
from jax.experimental.pallas import tpu as pltpu


def _gcn_kernel(adj_ref, x_ref, w_ref, wself_ref, b_ref, out_ref,
                support_ref, *, bm):
    m = pl.program_id(0)

    @pl.when(m == 0)
    def _():
        support_ref[...] = jnp.dot(x_ref[...], w_ref[...],
                                   preferred_element_type=jnp.float32)

    x_blk = x_ref[pl.ds(m * bm, bm), :]
    agg = jnp.dot(adj_ref[...], support_ref[...],
                  preferred_element_type=jnp.float32)
    self_part = jnp.dot(x_blk, wself_ref[...],
                        preferred_element_type=jnp.float32)
    out_ref[...] = agg + self_part + b_ref[...]


def kernel(input, adj, adj_homo, W, W_self, b):
    x = input.astype(jnp.float32)
    adj_homo = adj_homo.astype(jnp.float32)
    N, din = x.shape
    dout = W.shape[1]
    b2d = b.reshape(1, dout).astype(jnp.float32)

    BM = 400
    nm = N // BM

    import functools
    out = pl.pallas_call(
        functools.partial(_gcn_kernel, bm=BM),
        grid=(nm,),
        in_specs=[
            pl.BlockSpec((BM, N), lambda m: (m, 0)),
            pl.BlockSpec((N, din), lambda m: (0, 0)),
            pl.BlockSpec((din, dout), lambda m: (0, 0)),
            pl.BlockSpec((din, dout), lambda m: (0, 0)),
            pl.BlockSpec((1, dout), lambda m: (0, 0)),
        ],
        out_specs=pl.BlockSpec((BM, dout), lambda m: (m, 0)),
        out_shape=jax.ShapeDtypeStruct((N, dout), jnp.float32),
        scratch_shapes=[pltpu.VMEM((N, dout), jnp.float32)],
        compiler_params=pltpu.CompilerParams(
            dimension_semantics=("arbitrary",)),
    )(adj_homo, x, W.astype(jnp.float32), W_self.astype(jnp.float32), b2d)
    return out
